# baseline (speedup 1.0000x reference)
import functools

import jax
import jax.numpy as jnp
from jax import lax
from jax.experimental import pallas as pl
from jax.experimental.pallas import tpu as pltpu
from jax.experimental.pallas import tpu_sc as plsc

N = 10000
NPAD = 10240
E = 320000
P = 64000
HID = 128
NC = 2
NS = 16
B = 512
GRID = NPAD // B


def _sc_mesh():
    return plsc.VectorSubcoreMesh(
        core_axis_name="c", subcore_axis_name="s", num_cores=NC, num_subcores=NS)


def _deg_call(src, dst, ew, pb, pw):
    NW = NC * NS
    EPT = E // NW
    PPT = P // NW
    RPT = NPAD // NS

    @functools.partial(
        pl.kernel,
        out_type=(jax.ShapeDtypeStruct((NC, NPAD), jnp.float32),
                  jax.ShapeDtypeStruct((NC, NPAD), jnp.float32)),
        mesh=_sc_mesh(),
        scratch_types=[
            pltpu.VMEM((EPT,), jnp.int32),
            pltpu.VMEM((EPT,), jnp.int32),
            pltpu.VMEM((EPT,), jnp.float32),
            pltpu.VMEM((PPT,), jnp.int32),
            pltpu.VMEM((PPT,), jnp.float32),
            pltpu.VMEM((RPT,), jnp.float32),
            pltpu.VMEM_SHARED((NPAD,), jnp.float32),
            pltpu.VMEM_SHARED((NPAD,), jnp.float32),
        ],
    )
    def deg_k(src_hbm, dst_hbm, ew_hbm, pb_hbm, pw_hbm, b2b_hbm, p2b_hbm,
              src_v, dst_v, w_v, pb_v, pw_v, zero_v, degb_sh, degp_sh):
        c = lax.axis_index("c")
        s = lax.axis_index("s")
        wid = c * NS + s

        def zbody(i, _):
            zero_v[pl.ds(i * 16, 16)] = jnp.zeros((16,), jnp.float32)
            return ()
        lax.fori_loop(0, RPT // 16, zbody, ())
        pltpu.sync_copy(zero_v, degb_sh.at[pl.ds(s * RPT, RPT)])
        pltpu.sync_copy(zero_v, degp_sh.at[pl.ds(s * RPT, RPT)])
        plsc.subcore_barrier()
        eb = wid * EPT
        pltpu.sync_copy(src_hbm.at[pl.ds(eb, EPT)], src_v)
        pltpu.sync_copy(dst_hbm.at[pl.ds(eb, EPT)], dst_v)
        pltpu.sync_copy(ew_hbm.at[pl.ds(eb, EPT)], w_v)
        pltpu.sync_copy(w_v, degb_sh.at[src_v], add=True)
        pltpu.sync_copy(w_v, degb_sh.at[dst_v], add=True)
        pbase = wid * PPT
        pltpu.sync_copy(pb_hbm.at[pl.ds(pbase, PPT)], pb_v)
        pltpu.sync_copy(pw_hbm.at[pl.ds(pbase, PPT)], pw_v)
        pltpu.sync_copy(pw_v, degp_sh.at[pb_v], add=True)
        plsc.subcore_barrier()

        @pl.when(s == 0)
        def _():
            pltpu.sync_copy(degb_sh, b2b_hbm.at[c])
            pltpu.sync_copy(degp_sh, p2b_hbm.at[c])

    return deg_k(src, dst, ew, pb, pw)


def _agg_call(h2, eidx, ewr):
    HH = HID // NC
    EPT = E // NS
    K = 160
    NCH = EPT // K
    RPT = NPAD // NS

    @functools.partial(
        pl.kernel,
        out_type=jax.ShapeDtypeStruct((NC, NPAD, HH), jnp.float32),
        mesh=_sc_mesh(),
        compiler_params=pltpu.CompilerParams(use_tc_tiling_on_sc=False),
        scratch_types=[
            [pltpu.VMEM((2 * K,), jnp.int32) for _ in range(2)],
            [pltpu.VMEM((K,), jnp.float32) for _ in range(2)],
            [pltpu.VMEM((2 * K, HH), jnp.float32) for _ in range(2)],
            [pltpu.VMEM((K,), jnp.int32) for _ in range(2)],
            [pltpu.VMEM((K,), jnp.int32) for _ in range(2)],
            pltpu.VMEM_SHARED((NPAD, HH), jnp.float32),
            [pltpu.SemaphoreType.DMA for _ in range(2)],
            [pltpu.SemaphoreType.DMA for _ in range(2)],
            [pltpu.SemaphoreType.DMA for _ in range(2)],
        ],
    )
    def agg_k(h_hbm, eidx_hbm, ewr_hbm, out_hbm,
              ebuf, wbuf, rows, ssrc_v, sdst_v, agg_sh, sem_in, sem_g, sem_s):
        c = lax.axis_index("c")
        s = lax.axis_index("s")

        def zrow(i, _):
            for j in range(HH // 16):
                rows[0][i, pl.ds(j * 16, 16)] = jnp.zeros((16,), jnp.float32)
            return ()
        lax.fori_loop(0, 2 * K, zrow, ())
        base_r = s * RPT

        def zcopy(i, _):
            pltpu.sync_copy(rows[0], agg_sh.at[pl.ds(base_r + i * 2 * K, 2 * K)])
            return ()
        lax.fori_loop(0, RPT // (2 * K), zcopy, ())
        plsc.subcore_barrier()

        def issue_in(t, b):
            pltpu.async_copy(eidx_hbm.at[s].at[t], ebuf[b], sem_in[b])
            pltpu.async_copy(ewr_hbm.at[s].at[t], wbuf[b], sem_in[b])

        def wait_in(b):
            pltpu.make_async_copy(eidx_hbm.at[s].at[0], ebuf[b], sem_in[b]).wait()
            pltpu.make_async_copy(ewr_hbm.at[s].at[0], wbuf[b], sem_in[b]).wait()

        def issue_gather(b):
            pltpu.async_copy(h_hbm.at[c].at[ebuf[b].at[pl.ds(0, 2 * K)]],
                             rows[b], sem_g[b])

        def wait_gather(b):
            pltpu.make_async_copy(h_hbm.at[c].at[ebuf[b].at[pl.ds(0, 2 * K)]],
                                  rows[b], sem_g[b]).wait()

        def issue_scatter(b):
            pltpu.async_copy(rows[b].at[pl.ds(0, K)], agg_sh.at[ssrc_v[b]],
                             sem_s[b], add=True)
            pltpu.async_copy(rows[b].at[pl.ds(K, K)], agg_sh.at[sdst_v[b]],
                             sem_s[b], add=True)

        def wait_scatter(b):
            pltpu.make_async_copy(rows[b].at[pl.ds(0, K)], agg_sh.at[ssrc_v[b]],
                                  sem_s[b]).wait()
            pltpu.make_async_copy(rows[b].at[pl.ds(K, K)], agg_sh.at[sdst_v[b]],
                                  sem_s[b]).wait()

        def do_chunk(t, b, first=False, has_next=True, has_next2=True):
            wait_gather(b)
            for g in range(K // 16):
                sl = pl.ds(g * 16, 16)
                sdst_v[b][sl] = ebuf[b][sl]
            for g in range(K // 16):
                ssrc_v[b][pl.ds(g * 16, 16)] = ebuf[b][pl.ds(K + g * 16, 16)]
            if not first:
                wait_scatter(1 - b)
            if has_next:
                wait_in(1 - b)
                issue_gather(1 - b)

            def scale(g, _):
                w16 = wbuf[b][pl.ds(g * 16, 16)]
                base = g * 16
                for u in range(16):
                    wspl = lax.broadcast_in_dim(w16[u], (16,), ())
                    k = base + u
                    for j in range(HH // 16):
                        sl = pl.ds(j * 16, 16)
                        rows[b][k, sl] = rows[b][k, sl] * wspl
                        rows[b][K + k, sl] = rows[b][K + k, sl] * wspl
                return ()
            lax.fori_loop(0, K // 16, scale, ())
            issue_scatter(b)
            if has_next2:
                issue_in(t + 2, b)

        issue_in(0, 0)
        wait_in(0)
        issue_gather(0)
        issue_in(1, 1)
        do_chunk(0, 0, first=True)
        do_chunk(1, 1)

        def pair(i, _):
            t = 2 + i * 2
            do_chunk(t, 0)
            do_chunk(t + 1, 1)
            return ()
        lax.fori_loop(0, (NCH - 5) // 2, pair, ())
        do_chunk(NCH - 3, 0)
        do_chunk(NCH - 2, 1, has_next2=False)
        do_chunk(NCH - 1, 0, has_next=False, has_next2=False)

        wait_scatter((NCH - 1) % 2)
        plsc.subcore_barrier()
        pltpu.sync_copy(agg_sh.at[pl.ds(base_r, RPT)],
                        out_hbm.at[c, pl.ds(base_r, RPT)])

    return agg_k(h2, eidx, ewr)


def _mlp1_body(feats_ref, degb_ref, degp_ref, rid_ref, role_table_ref,
               W1a_ref, W1d_ref, W1r_ref, b1_ref, W2_ref, b2_ref, h_ref):
    feats = feats_ref[...]
    degc = (degb_ref[0] + degb_ref[1]
            + degp_ref[0] + degp_ref[1])
    oh = (rid_ref[...] == lax.broadcasted_iota(jnp.int32, (B, 8), 1)
          ).astype(jnp.float32)
    rproj = jnp.dot(role_table_ref[...], W1r_ref[...],
                    preferred_element_type=jnp.float32)
    pre = (jnp.dot(feats, W1a_ref[...], preferred_element_type=jnp.float32)
           + degc * W1d_ref[...]
           + jnp.dot(oh, rproj, preferred_element_type=jnp.float32)
           + b1_ref[...])
    h1 = jnp.maximum(pre, 0.0)
    h = jnp.dot(h1, W2_ref[...], preferred_element_type=jnp.float32) + b2_ref[...]
    h_ref[...] = jnp.maximum(h, 0.0)


def _mlp1_call(feats, degb, degp, rid, role_table, W1a, W1d, W1r, b1, W2, b2):
    full = lambda shape: pl.BlockSpec(shape, lambda i: (0,) * len(shape))
    row = lambda w: pl.BlockSpec((B, w), lambda i: (i, 0))
    return pl.pallas_call(
        _mlp1_body,
        grid=(GRID,),
        in_specs=[row(6),
                  pl.BlockSpec((NC, B, 1), lambda i: (0, i, 0)),
                  pl.BlockSpec((NC, B, 1), lambda i: (0, i, 0)),
                  row(1), full((8, 16)),
                  full((6, HID)), full((1, HID)), full((16, HID)),
                  full((1, HID)), full((HID, HID)), full((1, HID))],
        out_specs=row(HID),
        out_shape=jax.ShapeDtypeStruct((NPAD, HID), jnp.float32),
    )(feats, degb, degp, rid, role_table, W1a, W1d, W1r, b1, W2, b2)


def _mlp2_body(agg2_ref, h_ref, degp_ref, W3_ref, b3_ref, W4_ref, b4_ref,
               gamma_ref, beta_ref, be_ref, gsum_ref):
    i = pl.program_id(0)
    h = h_ref[...]
    agg = (jnp.concatenate([agg2_ref[0], agg2_ref[1]], axis=-1)
           + h * (degp_ref[0] + degp_ref[1]))
    t = jnp.maximum(
        jnp.dot(agg, W3_ref[...], preferred_element_type=jnp.float32)
        + b3_ref[...], 0.0)
    m = jnp.dot(t, W4_ref[...], preferred_element_type=jnp.float32) + b4_ref[...]
    y = h + m
    mu = jnp.mean(y, axis=-1, keepdims=True)
    var = jnp.mean((y - mu) ** 2, axis=-1, keepdims=True)
    be = (y - mu) * lax.rsqrt(var + 1e-5) * gamma_ref[...] + beta_ref[...]
    be_ref[...] = be

    rows = i * B + lax.broadcasted_iota(jnp.int32, (B, 1), 0)
    part = jnp.sum(jnp.where(rows < N, be, 0.0), axis=0, keepdims=True)

    @pl.when(i == 0)
    def _():
        gsum_ref[...] = jnp.zeros_like(gsum_ref)
    gsum_ref[...] += part

    @pl.when(i == pl.num_programs(0) - 1)
    def _():
        gsum_ref[...] = gsum_ref[...] * (1.0 / N)


def _mlp2_call(agg2, h, degp, W3, b3, W4, b4, gamma, beta):
    full = lambda shape: pl.BlockSpec(shape, lambda i: (0,) * len(shape))
    row = lambda w: pl.BlockSpec((B, w), lambda i: (i, 0))
    return pl.pallas_call(
        _mlp2_body,
        grid=(GRID,),
        in_specs=[pl.BlockSpec((NC, B, HID // NC), lambda i: (0, i, 0)),
                  row(HID),
                  pl.BlockSpec((NC, B, 1), lambda i: (0, i, 0)),
                  full((HID, HID)), full((1, HID)),
                  full((HID, HID)), full((1, HID)), full((1, HID)),
                  full((1, HID))],
        out_specs=[row(HID), pl.BlockSpec((1, HID), lambda i: (0, 0))],
        out_shape=[jax.ShapeDtypeStruct((NPAD, HID), jnp.float32),
                   jax.ShapeDtypeStruct((1, HID), jnp.float32)],
    )(agg2, h, degp, W3, b3, W4, b4, gamma, beta)


def kernel(area_targets, constraints, edge_index, edge_weight, p2b_block,
           p2b_weight, role_ids, role_table, W1, b1, W2, b2, W3, b3, W4, b4,
           gamma, beta):
    f32 = jnp.float32
    src = edge_index[0].astype(jnp.int32)
    dst = edge_index[1].astype(jnp.int32)
    pb = p2b_block.astype(jnp.int32)
    ew = edge_weight.astype(f32)
    pw = p2b_weight.astype(f32)

    degb, degp = _deg_call(src, dst, ew, pb, pw)

    feats = jnp.concatenate([area_targets[:, None], constraints], axis=1)
    feats = jnp.pad(feats, ((0, NPAD - N), (0, 0)))
    rid = jnp.pad(role_ids.astype(jnp.int32)[:, None], ((0, NPAD - N), (0, 0)))
    h = _mlp1_call(feats, degb[:, :, None], degp[:, :, None], rid, role_table,
                   W1[0:6], W1[6:7], W1[12:28], b1[None], W2, b2[None])

    h2 = jnp.stack([h[:, :HID // NC], h[:, HID // NC:]])
    KCH = 160
    NCHT = E // NS // KCH
    eidx = jnp.concatenate([dst.reshape(NS, NCHT, KCH),
                            src.reshape(NS, NCHT, KCH)], axis=2)
    agg2 = _agg_call(h2, eidx, ew.reshape(NS, NCHT, KCH))

    be_pad, gsum = _mlp2_call(agg2, h, degp[:, :, None], W3, b3[None], W4,
                              b4[None], gamma[None], beta[None])
    return be_pad[:N], gsum[0], jnp.ones((N,), dtype=bool)


# speedup vs baseline: 1.0339x; 1.0339x over previous
import functools

import jax
import jax.numpy as jnp
from jax import lax
from jax.experimental import pallas as pl
from jax.experimental.pallas import tpu as pltpu
from jax.experimental.pallas import tpu_sc as plsc

N = 10000
NPAD = 10240
E = 320000
P = 64000
HID = 128
NC = 2
NS = 16
B = 512
GRID = NPAD // B


def _sc_mesh():
    return plsc.VectorSubcoreMesh(
        core_axis_name="c", subcore_axis_name="s", num_cores=NC, num_subcores=NS)


def _deg_call(src, dst, ew, pb, pw):
    NW = NC * NS
    EPT = E // NW
    PPT = P // NW
    RPT = NPAD // NS

    @functools.partial(
        pl.kernel,
        out_type=(jax.ShapeDtypeStruct((NC, NPAD), jnp.float32),
                  jax.ShapeDtypeStruct((NC, NPAD), jnp.float32)),
        mesh=_sc_mesh(),
        scratch_types=[
            pltpu.VMEM((EPT,), jnp.int32),
            pltpu.VMEM((EPT,), jnp.int32),
            pltpu.VMEM((EPT,), jnp.float32),
            pltpu.VMEM((PPT,), jnp.int32),
            pltpu.VMEM((PPT,), jnp.float32),
            pltpu.VMEM((RPT,), jnp.float32),
            pltpu.VMEM_SHARED((NPAD,), jnp.float32),
            pltpu.VMEM_SHARED((NPAD,), jnp.float32),
        ],
    )
    def deg_k(src_hbm, dst_hbm, ew_hbm, pb_hbm, pw_hbm, b2b_hbm, p2b_hbm,
              src_v, dst_v, w_v, pb_v, pw_v, zero_v, degb_sh, degp_sh):
        c = lax.axis_index("c")
        s = lax.axis_index("s")
        wid = c * NS + s

        def zbody(i, _):
            zero_v[pl.ds(i * 16, 16)] = jnp.zeros((16,), jnp.float32)
            return ()
        lax.fori_loop(0, RPT // 16, zbody, ())
        pltpu.sync_copy(zero_v, degb_sh.at[pl.ds(s * RPT, RPT)])
        pltpu.sync_copy(zero_v, degp_sh.at[pl.ds(s * RPT, RPT)])
        plsc.subcore_barrier()
        eb = wid * EPT
        pltpu.sync_copy(src_hbm.at[pl.ds(eb, EPT)], src_v)
        pltpu.sync_copy(dst_hbm.at[pl.ds(eb, EPT)], dst_v)
        pltpu.sync_copy(ew_hbm.at[pl.ds(eb, EPT)], w_v)
        pltpu.sync_copy(w_v, degb_sh.at[src_v], add=True)
        pltpu.sync_copy(w_v, degb_sh.at[dst_v], add=True)
        pbase = wid * PPT
        pltpu.sync_copy(pb_hbm.at[pl.ds(pbase, PPT)], pb_v)
        pltpu.sync_copy(pw_hbm.at[pl.ds(pbase, PPT)], pw_v)
        pltpu.sync_copy(pw_v, degp_sh.at[pb_v], add=True)
        plsc.subcore_barrier()

        @pl.when(s == 0)
        def _():
            pltpu.sync_copy(degb_sh, b2b_hbm.at[c])
            pltpu.sync_copy(degp_sh, p2b_hbm.at[c])

    return deg_k(src, dst, ew, pb, pw)


def _agg_call(h2, dst3, src3, ewr):
    HH = HID // NC
    EPT = E // NS
    K = 160
    NCH = EPT // K
    RPT = NPAD // NS

    @functools.partial(
        pl.kernel,
        out_type=jax.ShapeDtypeStruct((NC, NPAD, HH), jnp.float32),
        mesh=_sc_mesh(),
        compiler_params=pltpu.CompilerParams(use_tc_tiling_on_sc=False),
        scratch_types=[
            [pltpu.VMEM((2 * K,), jnp.int32) for _ in range(2)],
            [pltpu.VMEM((K,), jnp.float32) for _ in range(2)],
            [pltpu.VMEM((2 * K, HH), jnp.float32) for _ in range(2)],
            [pltpu.VMEM((K,), jnp.int32) for _ in range(2)],
            [pltpu.VMEM((K,), jnp.int32) for _ in range(2)],
            pltpu.VMEM_SHARED((NPAD, HH), jnp.float32),
            [pltpu.SemaphoreType.DMA for _ in range(2)],
            [pltpu.SemaphoreType.DMA for _ in range(2)],
            [pltpu.SemaphoreType.DMA for _ in range(2)],
        ],
    )
    def agg_k(h_hbm, dst_hbm, src_hbm, ewr_hbm, out_hbm,
              ebuf, wbuf, rows, ssrc_v, sdst_v, agg_sh, sem_in, sem_g, sem_s):
        c = lax.axis_index("c")
        s = lax.axis_index("s")

        def zrow(i, _):
            for j in range(HH // 16):
                rows[0][i, pl.ds(j * 16, 16)] = jnp.zeros((16,), jnp.float32)
            return ()
        lax.fori_loop(0, 2 * K, zrow, ())
        base_r = s * RPT

        def zcopy(i, _):
            pltpu.sync_copy(rows[0], agg_sh.at[pl.ds(base_r + i * 2 * K, 2 * K)])
            return ()
        lax.fori_loop(0, RPT // (2 * K), zcopy, ())
        plsc.subcore_barrier()

        def issue_in(t, b):
            pltpu.async_copy(dst_hbm.at[s].at[t], ebuf[b].at[pl.ds(0, K)],
                             sem_in[b])
            pltpu.async_copy(src_hbm.at[s].at[t], ebuf[b].at[pl.ds(K, K)],
                             sem_in[b])
            pltpu.async_copy(ewr_hbm.at[s].at[t], wbuf[b], sem_in[b])

        def wait_in(b):
            pltpu.make_async_copy(dst_hbm.at[s].at[0], ebuf[b].at[pl.ds(0, K)],
                                  sem_in[b]).wait()
            pltpu.make_async_copy(src_hbm.at[s].at[0], ebuf[b].at[pl.ds(K, K)],
                                  sem_in[b]).wait()
            pltpu.make_async_copy(ewr_hbm.at[s].at[0], wbuf[b], sem_in[b]).wait()

        def issue_gather(b):
            pltpu.async_copy(h_hbm.at[c].at[ebuf[b].at[pl.ds(0, 2 * K)]],
                             rows[b], sem_g[b])

        def wait_gather(b):
            pltpu.make_async_copy(h_hbm.at[c].at[ebuf[b].at[pl.ds(0, 2 * K)]],
                                  rows[b], sem_g[b]).wait()

        def issue_scatter(b):
            pltpu.async_copy(rows[b].at[pl.ds(0, K)], agg_sh.at[ssrc_v[b]],
                             sem_s[b], add=True)
            pltpu.async_copy(rows[b].at[pl.ds(K, K)], agg_sh.at[sdst_v[b]],
                             sem_s[b], add=True)

        def wait_scatter(b):
            pltpu.make_async_copy(rows[b].at[pl.ds(0, K)], agg_sh.at[ssrc_v[b]],
                                  sem_s[b]).wait()
            pltpu.make_async_copy(rows[b].at[pl.ds(K, K)], agg_sh.at[sdst_v[b]],
                                  sem_s[b]).wait()

        def do_chunk(t, b, first=False, has_next=True, has_next2=True):
            wait_gather(b)
            for g in range(K // 16):
                sl = pl.ds(g * 16, 16)
                sdst_v[b][sl] = ebuf[b][sl]
            for g in range(K // 16):
                ssrc_v[b][pl.ds(g * 16, 16)] = ebuf[b][pl.ds(K + g * 16, 16)]
            if not first:
                wait_scatter(1 - b)
            if has_next:
                wait_in(1 - b)
                issue_gather(1 - b)

            def scale(g, _):
                w16 = wbuf[b][pl.ds(g * 16, 16)]
                base = g * 16
                for u in range(16):
                    wspl = lax.broadcast_in_dim(w16[u], (16,), ())
                    k = base + u
                    for j in range(HH // 16):
                        sl = pl.ds(j * 16, 16)
                        rows[b][k, sl] = rows[b][k, sl] * wspl
                        rows[b][K + k, sl] = rows[b][K + k, sl] * wspl
                return ()
            lax.fori_loop(0, K // 16, scale, ())
            issue_scatter(b)
            if has_next2:
                issue_in(t + 2, b)

        issue_in(0, 0)
        wait_in(0)
        issue_gather(0)
        issue_in(1, 1)
        do_chunk(0, 0, first=True)
        do_chunk(1, 1)

        def pair(i, _):
            t = 2 + i * 2
            do_chunk(t, 0)
            do_chunk(t + 1, 1)
            return ()
        lax.fori_loop(0, (NCH - 5) // 2, pair, ())
        do_chunk(NCH - 3, 0)
        do_chunk(NCH - 2, 1, has_next2=False)
        do_chunk(NCH - 1, 0, has_next=False, has_next2=False)

        wait_scatter((NCH - 1) % 2)
        plsc.subcore_barrier()
        pltpu.sync_copy(agg_sh.at[pl.ds(base_r, RPT)],
                        out_hbm.at[c, pl.ds(base_r, RPT)])

    return agg_k(h2, dst3, src3, ewr)


def _mlp1_body(feats_ref, degb_ref, degp_ref, rid_ref, role_table_ref,
               W1a_ref, W1d_ref, W1r_ref, b1_ref, W2_ref, b2_ref, h_ref):
    feats = feats_ref[...]
    degc = (degb_ref[0] + degb_ref[1]
            + degp_ref[0] + degp_ref[1])
    oh = (rid_ref[...] == lax.broadcasted_iota(jnp.int32, (B, 8), 1)
          ).astype(jnp.float32)
    rproj = jnp.dot(role_table_ref[...], W1r_ref[...],
                    preferred_element_type=jnp.float32)
    pre = (jnp.dot(feats, W1a_ref[...], preferred_element_type=jnp.float32)
           + degc * W1d_ref[...]
           + jnp.dot(oh, rproj, preferred_element_type=jnp.float32)
           + b1_ref[...])
    h1 = jnp.maximum(pre, 0.0)
    h = jnp.dot(h1, W2_ref[...], preferred_element_type=jnp.float32) + b2_ref[...]
    h_ref[...] = jnp.maximum(h, 0.0)


def _mlp1_call(feats, degb, degp, rid, role_table, W1a, W1d, W1r, b1, W2, b2):
    full = lambda shape: pl.BlockSpec(shape, lambda i: (0,) * len(shape))
    row = lambda w: pl.BlockSpec((B, w), lambda i: (i, 0))
    return pl.pallas_call(
        _mlp1_body,
        grid=(GRID,),
        in_specs=[row(6),
                  pl.BlockSpec((NC, B, 1), lambda i: (0, i, 0)),
                  pl.BlockSpec((NC, B, 1), lambda i: (0, i, 0)),
                  row(1), full((8, 16)),
                  full((6, HID)), full((1, HID)), full((16, HID)),
                  full((1, HID)), full((HID, HID)), full((1, HID))],
        out_specs=row(HID),
        out_shape=jax.ShapeDtypeStruct((NPAD, HID), jnp.float32),
    )(feats, degb, degp, rid, role_table, W1a, W1d, W1r, b1, W2, b2)


def _mlp2_body(agg2_ref, h_ref, degp_ref, W3_ref, b3_ref, W4_ref, b4_ref,
               gamma_ref, beta_ref, be_ref, gsum_ref):
    i = pl.program_id(0)
    h = h_ref[...]
    agg = (jnp.concatenate([agg2_ref[0], agg2_ref[1]], axis=-1)
           + h * (degp_ref[0] + degp_ref[1]))
    t = jnp.maximum(
        jnp.dot(agg, W3_ref[...], preferred_element_type=jnp.float32)
        + b3_ref[...], 0.0)
    m = jnp.dot(t, W4_ref[...], preferred_element_type=jnp.float32) + b4_ref[...]
    y = h + m
    mu = jnp.mean(y, axis=-1, keepdims=True)
    var = jnp.mean((y - mu) ** 2, axis=-1, keepdims=True)
    be = (y - mu) * lax.rsqrt(var + 1e-5) * gamma_ref[...] + beta_ref[...]
    be_ref[...] = be

    rows = i * B + lax.broadcasted_iota(jnp.int32, (B, 1), 0)
    part = jnp.sum(jnp.where(rows < N, be, 0.0), axis=0, keepdims=True)

    @pl.when(i == 0)
    def _():
        gsum_ref[...] = jnp.zeros_like(gsum_ref)
    gsum_ref[...] += part

    @pl.when(i == pl.num_programs(0) - 1)
    def _():
        gsum_ref[...] = gsum_ref[...] * (1.0 / N)


def _mlp2_call(agg2, h, degp, W3, b3, W4, b4, gamma, beta):
    full = lambda shape: pl.BlockSpec(shape, lambda i: (0,) * len(shape))
    row = lambda w: pl.BlockSpec((B, w), lambda i: (i, 0))
    return pl.pallas_call(
        _mlp2_body,
        grid=(GRID,),
        in_specs=[pl.BlockSpec((NC, B, HID // NC), lambda i: (0, i, 0)),
                  row(HID),
                  pl.BlockSpec((NC, B, 1), lambda i: (0, i, 0)),
                  full((HID, HID)), full((1, HID)),
                  full((HID, HID)), full((1, HID)), full((1, HID)),
                  full((1, HID))],
        out_specs=[row(HID), pl.BlockSpec((1, HID), lambda i: (0, 0))],
        out_shape=[jax.ShapeDtypeStruct((NPAD, HID), jnp.float32),
                   jax.ShapeDtypeStruct((1, HID), jnp.float32)],
    )(agg2, h, degp, W3, b3, W4, b4, gamma, beta)


def kernel(area_targets, constraints, edge_index, edge_weight, p2b_block,
           p2b_weight, role_ids, role_table, W1, b1, W2, b2, W3, b3, W4, b4,
           gamma, beta):
    f32 = jnp.float32
    src = edge_index[0].astype(jnp.int32)
    dst = edge_index[1].astype(jnp.int32)
    pb = p2b_block.astype(jnp.int32)
    ew = edge_weight.astype(f32)
    pw = p2b_weight.astype(f32)

    degb, degp = _deg_call(src, dst, ew, pb, pw)

    feats = jnp.concatenate([area_targets[:, None], constraints], axis=1)
    feats = jnp.pad(feats, ((0, NPAD - N), (0, 0)))
    rid = jnp.pad(role_ids.astype(jnp.int32)[:, None], ((0, NPAD - N), (0, 0)))
    h = _mlp1_call(feats, degb[:, :, None], degp[:, :, None], rid, role_table,
                   W1[0:6], W1[6:7], W1[12:28], b1[None], W2, b2[None])

    h2 = jnp.stack([h[:, :HID // NC], h[:, HID // NC:]])
    KCH = 160
    NCHT = E // NS // KCH
    agg2 = _agg_call(h2, dst.reshape(NS, NCHT, KCH),
                     src.reshape(NS, NCHT, KCH), ew.reshape(NS, NCHT, KCH))

    be_pad, gsum = _mlp2_call(agg2, h, degp[:, :, None], W3, b3[None], W4,
                              b4[None], gamma[None], beta[None])
    return be_pad[:N], gsum[0], jnp.ones((N,), dtype=bool)


# speedup vs baseline: 1.0541x; 1.0195x over previous
import functools

import jax
import jax.numpy as jnp
from jax import lax
from jax.experimental import pallas as pl
from jax.experimental.pallas import tpu as pltpu
from jax.experimental.pallas import tpu_sc as plsc

N = 10000
NPAD = 10240
E = 320000
P = 64000
HID = 128
NC = 2
NS = 16
B = 512
GRID = NPAD // B


def _sc_mesh():
    return plsc.VectorSubcoreMesh(
        core_axis_name="c", subcore_axis_name="s", num_cores=NC, num_subcores=NS)


def _deg_call(src, dst, ew, pb, pw):
    NW = NC * NS
    EPT = E // NW
    PPT = P // NW
    RPT = NPAD // NS

    @functools.partial(
        pl.kernel,
        out_type=(jax.ShapeDtypeStruct((NC, NPAD), jnp.float32),
                  jax.ShapeDtypeStruct((NC, NPAD), jnp.float32)),
        mesh=_sc_mesh(),
        scratch_types=[
            pltpu.VMEM((EPT,), jnp.int32),
            pltpu.VMEM((EPT,), jnp.int32),
            pltpu.VMEM((EPT,), jnp.float32),
            pltpu.VMEM((PPT,), jnp.int32),
            pltpu.VMEM((PPT,), jnp.float32),
            pltpu.VMEM((RPT,), jnp.float32),
            pltpu.VMEM_SHARED((NPAD,), jnp.float32),
            pltpu.VMEM_SHARED((NPAD,), jnp.float32),
        ],
    )
    def deg_k(src_hbm, dst_hbm, ew_hbm, pb_hbm, pw_hbm, b2b_hbm, p2b_hbm,
              src_v, dst_v, w_v, pb_v, pw_v, zero_v, degb_sh, degp_sh):
        c = lax.axis_index("c")
        s = lax.axis_index("s")
        wid = c * NS + s

        def zbody(i, _):
            zero_v[pl.ds(i * 16, 16)] = jnp.zeros((16,), jnp.float32)
            return ()
        lax.fori_loop(0, RPT // 16, zbody, ())
        pltpu.sync_copy(zero_v, degb_sh.at[pl.ds(s * RPT, RPT)])
        pltpu.sync_copy(zero_v, degp_sh.at[pl.ds(s * RPT, RPT)])
        plsc.subcore_barrier()
        eb = wid * EPT
        pltpu.sync_copy(src_hbm.at[pl.ds(eb, EPT)], src_v)
        pltpu.sync_copy(dst_hbm.at[pl.ds(eb, EPT)], dst_v)
        pltpu.sync_copy(ew_hbm.at[pl.ds(eb, EPT)], w_v)
        pltpu.sync_copy(w_v, degb_sh.at[src_v], add=True)
        pltpu.sync_copy(w_v, degb_sh.at[dst_v], add=True)
        pbase = wid * PPT
        pltpu.sync_copy(pb_hbm.at[pl.ds(pbase, PPT)], pb_v)
        pltpu.sync_copy(pw_hbm.at[pl.ds(pbase, PPT)], pw_v)
        pltpu.sync_copy(pw_v, degp_sh.at[pb_v], add=True)
        plsc.subcore_barrier()

        @pl.when(s == 0)
        def _():
            pltpu.sync_copy(degb_sh, b2b_hbm.at[c])
            pltpu.sync_copy(degp_sh, p2b_hbm.at[c])

    return deg_k(src, dst, ew, pb, pw)


def _agg_call(h2, dst3, src3, ewr):
    HH = HID // NC
    EPT = E // NS
    K = 160
    NCH = EPT // K
    RPT = NPAD // NS

    @functools.partial(
        pl.kernel,
        out_type=jax.ShapeDtypeStruct((NC, NPAD, HH), jnp.float32),
        mesh=_sc_mesh(),
        compiler_params=pltpu.CompilerParams(use_tc_tiling_on_sc=False),
        scratch_types=[
            [pltpu.VMEM((2 * K,), jnp.int32) for _ in range(2)],
            [pltpu.VMEM((K,), jnp.float32) for _ in range(2)],
            [pltpu.VMEM((2 * K, HH), jnp.float32) for _ in range(2)],
            [pltpu.VMEM((K,), jnp.int32) for _ in range(2)],
            [pltpu.VMEM((K,), jnp.int32) for _ in range(2)],
            pltpu.VMEM_SHARED((NPAD, HH), jnp.float32),
            [pltpu.SemaphoreType.DMA for _ in range(2)],
            [pltpu.SemaphoreType.DMA for _ in range(2)],
            [pltpu.SemaphoreType.DMA for _ in range(2)],
        ],
    )
    def agg_k(h_hbm, dst_hbm, src_hbm, ewr_hbm, out_hbm,
              ebuf, wbuf, rows, ssrc_v, sdst_v, agg_sh, sem_in, sem_g, sem_s):
        c = lax.axis_index("c")
        s = lax.axis_index("s")

        def zrow(i, _):
            for j in range(HH // 16):
                rows[0][i, pl.ds(j * 16, 16)] = jnp.zeros((16,), jnp.float32)
            return ()
        lax.fori_loop(0, 2 * K, zrow, ())
        base_r = s * RPT

        def zcopy(i, _):
            pltpu.sync_copy(rows[0], agg_sh.at[pl.ds(base_r + i * 2 * K, 2 * K)])
            return ()
        lax.fori_loop(0, RPT // (2 * K), zcopy, ())
        plsc.subcore_barrier()

        def issue_in(t, b):
            pltpu.async_copy(dst_hbm.at[s].at[t], ebuf[b].at[pl.ds(0, K)],
                             sem_in[b])
            pltpu.async_copy(src_hbm.at[s].at[t], ebuf[b].at[pl.ds(K, K)],
                             sem_in[b])
            pltpu.async_copy(ewr_hbm.at[s].at[t], wbuf[b], sem_in[b])

        def wait_in(b):
            pltpu.make_async_copy(dst_hbm.at[s].at[0], ebuf[b].at[pl.ds(0, K)],
                                  sem_in[b]).wait()
            pltpu.make_async_copy(src_hbm.at[s].at[0], ebuf[b].at[pl.ds(K, K)],
                                  sem_in[b]).wait()
            pltpu.make_async_copy(ewr_hbm.at[s].at[0], wbuf[b], sem_in[b]).wait()

        def issue_gather(b):
            pltpu.async_copy(h_hbm.at[c].at[ebuf[b].at[pl.ds(0, 2 * K)]],
                             rows[b], sem_g[b])

        def wait_gather(b):
            pltpu.make_async_copy(h_hbm.at[c].at[ebuf[b].at[pl.ds(0, 2 * K)]],
                                  rows[b], sem_g[b]).wait()

        def issue_scatter(b):
            pltpu.async_copy(rows[b].at[pl.ds(0, K)], agg_sh.at[ssrc_v[b]],
                             sem_s[b], add=True)
            pltpu.async_copy(rows[b].at[pl.ds(K, K)], agg_sh.at[sdst_v[b]],
                             sem_s[b], add=True)

        def wait_scatter(b):
            pltpu.make_async_copy(rows[b].at[pl.ds(0, K)], agg_sh.at[ssrc_v[b]],
                                  sem_s[b]).wait()
            pltpu.make_async_copy(rows[b].at[pl.ds(K, K)], agg_sh.at[sdst_v[b]],
                                  sem_s[b]).wait()

        def do_chunk(t, b, first=False, has_next=True, has_next2=True):
            wait_gather(b)
            for g in range(K // 16):
                sl = pl.ds(g * 16, 16)
                sdst_v[b][sl] = ebuf[b][sl]
            for g in range(K // 16):
                ssrc_v[b][pl.ds(g * 16, 16)] = ebuf[b][pl.ds(K + g * 16, 16)]
            if not first:
                wait_scatter(1 - b)
            if has_next:
                wait_in(1 - b)
                issue_gather(1 - b)

            def scale(g, _):
                w16 = wbuf[b][pl.ds(g * 16, 16)]
                base = g * 16
                for u in range(16):
                    wspl = lax.broadcast_in_dim(w16[u], (16,), ())
                    k = base + u
                    for j in range(HH // 16):
                        sl = pl.ds(j * 16, 16)
                        rows[b][k, sl] = rows[b][k, sl] * wspl
                        rows[b][K + k, sl] = rows[b][K + k, sl] * wspl
                return ()
            lax.fori_loop(0, K // 16, scale, ())
            issue_scatter(b)
            if has_next2:
                issue_in(t + 2, b)

        issue_in(0, 0)
        wait_in(0)
        issue_gather(0)
        issue_in(1, 1)
        do_chunk(0, 0, first=True)
        do_chunk(1, 1)

        def pair(i, _):
            t = 2 + i * 2
            do_chunk(t, 0)
            do_chunk(t + 1, 1)
            return ()
        lax.fori_loop(0, (NCH - 5) // 2, pair, ())
        do_chunk(NCH - 3, 0)
        do_chunk(NCH - 2, 1, has_next2=False)
        do_chunk(NCH - 1, 0, has_next=False, has_next2=False)

        wait_scatter((NCH - 1) % 2)
        plsc.subcore_barrier()
        pltpu.sync_copy(agg_sh.at[pl.ds(base_r, RPT)],
                        out_hbm.at[c, pl.ds(base_r, RPT)])

    return agg_k(h2, dst3, src3, ewr)


def _mlp1_body(feats_ref, degb_ref, degp_ref, rid_ref, role_table_ref,
               W1a_ref, W1d_ref, W1r_ref, b1_ref, W2_ref, b2_ref, h_ref,
               h2_ref):
    feats = feats_ref[...]
    degc = (degb_ref[0] + degb_ref[1]
            + degp_ref[0] + degp_ref[1])
    oh = (rid_ref[...] == lax.broadcasted_iota(jnp.int32, (B, 8), 1)
          ).astype(jnp.float32)
    rproj = jnp.dot(role_table_ref[...], W1r_ref[...],
                    preferred_element_type=jnp.float32)
    pre = (jnp.dot(feats, W1a_ref[...], preferred_element_type=jnp.float32)
           + degc * W1d_ref[...]
           + jnp.dot(oh, rproj, preferred_element_type=jnp.float32)
           + b1_ref[...])
    h1 = jnp.maximum(pre, 0.0)
    h = jnp.dot(h1, W2_ref[...], preferred_element_type=jnp.float32) + b2_ref[...]
    h = jnp.maximum(h, 0.0)
    h_ref[...] = h
    h2_ref[0] = h[:, :HID // NC]
    h2_ref[1] = h[:, HID // NC:]


def _mlp1_call(feats, degb, degp, rid, role_table, W1a, W1d, W1r, b1, W2, b2):
    full = lambda shape: pl.BlockSpec(shape, lambda i: (0,) * len(shape))
    row = lambda w: pl.BlockSpec((B, w), lambda i: (i, 0))
    return pl.pallas_call(
        _mlp1_body,
        grid=(GRID,),
        in_specs=[row(6),
                  pl.BlockSpec((NC, B, 1), lambda i: (0, i, 0)),
                  pl.BlockSpec((NC, B, 1), lambda i: (0, i, 0)),
                  row(1), full((8, 16)),
                  full((6, HID)), full((1, HID)), full((16, HID)),
                  full((1, HID)), full((HID, HID)), full((1, HID))],
        out_specs=[row(HID),
                   pl.BlockSpec((NC, B, HID // NC), lambda i: (0, i, 0))],
        out_shape=[jax.ShapeDtypeStruct((NPAD, HID), jnp.float32),
                   jax.ShapeDtypeStruct((NC, NPAD, HID // NC), jnp.float32)],
    )(feats, degb, degp, rid, role_table, W1a, W1d, W1r, b1, W2, b2)


def _mlp2_body(agg2_ref, h_ref, degp_ref, W3_ref, b3_ref, W4_ref, b4_ref,
               gamma_ref, beta_ref, be_ref, gsum_ref):
    i = pl.program_id(0)
    h = h_ref[...]
    agg = (jnp.concatenate([agg2_ref[0], agg2_ref[1]], axis=-1)
           + h * (degp_ref[0] + degp_ref[1]))
    t = jnp.maximum(
        jnp.dot(agg, W3_ref[...], preferred_element_type=jnp.float32)
        + b3_ref[...], 0.0)
    m = jnp.dot(t, W4_ref[...], preferred_element_type=jnp.float32) + b4_ref[...]
    y = h + m
    mu = jnp.mean(y, axis=-1, keepdims=True)
    var = jnp.mean((y - mu) ** 2, axis=-1, keepdims=True)
    be = (y - mu) * lax.rsqrt(var + 1e-5) * gamma_ref[...] + beta_ref[...]
    be_ref[...] = be

    part = jnp.sum(be, axis=0, keepdims=True)

    @pl.when(i == 0)
    def _():
        gsum_ref[...] = jnp.zeros_like(gsum_ref)
    gsum_ref[...] += part

    @pl.when(i == pl.num_programs(0) - 1)
    def _():
        gsum_ref[...] = gsum_ref[...] * (1.0 / N)


def _mlp2_call(agg2, h, degp, W3, b3, W4, b4, gamma, beta):
    B2 = 400
    full = lambda shape: pl.BlockSpec(shape, lambda i: (0,) * len(shape))
    row = lambda w: pl.BlockSpec((B2, w), lambda i: (i, 0))
    return pl.pallas_call(
        _mlp2_body,
        grid=(N // B2,),
        in_specs=[pl.BlockSpec((NC, B2, HID // NC), lambda i: (0, i, 0)),
                  row(HID),
                  pl.BlockSpec((NC, B2, 1), lambda i: (0, i, 0)),
                  full((HID, HID)), full((1, HID)),
                  full((HID, HID)), full((1, HID)), full((1, HID)),
                  full((1, HID))],
        out_specs=[row(HID), pl.BlockSpec((1, HID), lambda i: (0, 0))],
        out_shape=[jax.ShapeDtypeStruct((N, HID), jnp.float32),
                   jax.ShapeDtypeStruct((1, HID), jnp.float32)],
    )(agg2, h, degp, W3, b3, W4, b4, gamma, beta)


def kernel(area_targets, constraints, edge_index, edge_weight, p2b_block,
           p2b_weight, role_ids, role_table, W1, b1, W2, b2, W3, b3, W4, b4,
           gamma, beta):
    f32 = jnp.float32
    src = edge_index[0].astype(jnp.int32)
    dst = edge_index[1].astype(jnp.int32)
    pb = p2b_block.astype(jnp.int32)
    ew = edge_weight.astype(f32)
    pw = p2b_weight.astype(f32)

    degb, degp = _deg_call(src, dst, ew, pb, pw)

    feats = jnp.concatenate([area_targets[:, None], constraints], axis=1)
    feats = jnp.pad(feats, ((0, NPAD - N), (0, 0)))
    rid = jnp.pad(role_ids.astype(jnp.int32)[:, None], ((0, NPAD - N), (0, 0)))
    h, h2 = _mlp1_call(feats, degb[:, :, None], degp[:, :, None], rid,
                       role_table, W1[0:6], W1[6:7], W1[12:28], b1[None], W2,
                       b2[None])

    KCH = 160
    NCHT = E // NS // KCH
    agg2 = _agg_call(h2, dst.reshape(NS, NCHT, KCH),
                     src.reshape(NS, NCHT, KCH), ew.reshape(NS, NCHT, KCH))

    be, gsum = _mlp2_call(agg2, h, degp[:, :, None], W3, b3[None], W4,
                          b4[None], gamma[None], beta[None])
    return be, gsum[0], jnp.ones((N,), dtype=bool)
